# initial kernel scaffold (unmeasured)
import jax
import jax.numpy as jnp
from jax import lax
from jax.experimental import pallas as pl
from jax.experimental.pallas import tpu as pltpu

M = 4096
D = 4096
MB = M // 4


def kernel(dy, W):
    x = lax.axis_index("x")
    z = lax.axis_index("z")
    b = 2 * x + z
    dy_b = lax.dynamic_slice_in_dim(dy, b * MB, MB, axis=0)
    P = lax.dot_general(
        dy_b, W, (((1,), (1,)), ((), ())), preferred_element_type=jnp.float32
    )

    def body(p_ref, out_ref, yrecv_ref, sems):
        xi = lax.axis_index("x")
        yi = lax.axis_index("y")
        zi = lax.axis_index("z")
        bb = 2 * xi + zi

        barrier = pltpu.get_barrier_semaphore()
        for nbr in [(1 - xi, yi, zi), (xi, 1 - yi, zi), (xi, yi, 1 - zi)]:
            pl.semaphore_signal(
                barrier, inc=1, device_id=nbr,
                device_id_type=pl.DeviceIdType.MESH,
            )
        pl.semaphore_wait(barrier, 3)

        y_rdma = pltpu.make_async_remote_copy(
            src_ref=p_ref,
            dst_ref=yrecv_ref,
            send_sem=sems.at[0],
            recv_sem=sems.at[1],
            device_id=(xi, 1 - yi, zi),
            device_id_type=pl.DeviceIdType.MESH,
        )
        y_rdma.start()
        y_rdma.wait()
        out_ref[pl.ds(bb * MB, MB), :] = p_ref[...] + yrecv_ref[...]

        z_rdma = pltpu.make_async_remote_copy(
            src_ref=out_ref.at[pl.ds(bb * MB, MB), :],
            dst_ref=out_ref.at[pl.ds(bb * MB, MB), :],
            send_sem=sems.at[2],
            recv_sem=sems.at[3],
            device_id=(xi, yi, 1 - zi),
            device_id_type=pl.DeviceIdType.MESH,
        )
        z_rdma.start()
        z_rdma.wait()

        x_rdma = pltpu.make_async_remote_copy(
            src_ref=out_ref.at[pl.ds(xi * 2 * MB, 2 * MB), :],
            dst_ref=out_ref.at[pl.ds(xi * 2 * MB, 2 * MB), :],
            send_sem=sems.at[4],
            recv_sem=sems.at[5],
            device_id=(1 - xi, yi, zi),
            device_id_type=pl.DeviceIdType.MESH,
        )
        x_rdma.start()
        x_rdma.wait()

    return pl.pallas_call(
        body,
        out_shape=jax.ShapeDtypeStruct((M, D), jnp.float32),
        in_specs=[pl.BlockSpec(memory_space=pltpu.VMEM)],
        out_specs=pl.BlockSpec(memory_space=pltpu.VMEM),
        scratch_shapes=[
            pltpu.VMEM((MB, D), jnp.float32),
            pltpu.SemaphoreType.DMA((6,)),
        ],
        compiler_params=pltpu.CompilerParams(collective_id=0),
    )(P)


# baseline (device time: 897185 ns/iter reference)
import jax
import jax.numpy as jnp
from jax import lax
from jax.experimental import pallas as pl
from jax.experimental.pallas import tpu as pltpu

M = 4096
D = 4096
MB = M // 4


def kernel(dy, W):
    x = lax.axis_index("x")
    z = lax.axis_index("z")
    b = 2 * x + z
    dy_b = lax.dynamic_slice_in_dim(dy, b * MB, MB, axis=0)
    P = lax.dot_general(
        dy_b, W, (((1,), (1,)), ((), ())), preferred_element_type=jnp.float32
    )

    def body(p_ref, out_ref, yrecv_ref, sems, copy_sem):
        xi = lax.axis_index("x")
        yi = lax.axis_index("y")
        zi = lax.axis_index("z")
        bb = 2 * xi + zi

        barrier = pltpu.get_barrier_semaphore()
        for nbr in [(1 - xi, yi, zi), (xi, 1 - yi, zi), (xi, yi, 1 - zi)]:
            pl.semaphore_signal(
                barrier, inc=1, device_id=nbr,
                device_id_type=pl.DeviceIdType.MESH,
            )
        pl.semaphore_wait(barrier, 3)

        y_rdma = pltpu.make_async_remote_copy(
            src_ref=p_ref,
            dst_ref=yrecv_ref,
            send_sem=sems.at[0],
            recv_sem=sems.at[1],
            device_id=(xi, 1 - yi, zi),
            device_id_type=pl.DeviceIdType.MESH,
        )
        y_rdma.start()
        y_rdma.wait()
        yrecv_ref[...] = p_ref[...] + yrecv_ref[...]
        store = pltpu.make_async_copy(
            yrecv_ref, out_ref.at[pl.ds(bb * MB, MB), :], copy_sem
        )
        store.start()
        store.wait()

        z_rdma = pltpu.make_async_remote_copy(
            src_ref=out_ref.at[pl.ds(bb * MB, MB), :],
            dst_ref=out_ref.at[pl.ds(bb * MB, MB), :],
            send_sem=sems.at[2],
            recv_sem=sems.at[3],
            device_id=(xi, yi, 1 - zi),
            device_id_type=pl.DeviceIdType.MESH,
        )
        z_rdma.start()
        z_rdma.wait()

        x_rdma = pltpu.make_async_remote_copy(
            src_ref=out_ref.at[pl.ds(xi * 2 * MB, 2 * MB), :],
            dst_ref=out_ref.at[pl.ds(xi * 2 * MB, 2 * MB), :],
            send_sem=sems.at[4],
            recv_sem=sems.at[5],
            device_id=(1 - xi, yi, zi),
            device_id_type=pl.DeviceIdType.MESH,
        )
        x_rdma.start()
        x_rdma.wait()

    return pl.pallas_call(
        body,
        out_shape=jax.ShapeDtypeStruct((M, D), jnp.float32),
        in_specs=[pl.BlockSpec(memory_space=pltpu.VMEM)],
        out_specs=pl.BlockSpec(memory_space=pl.ANY),
        scratch_shapes=[
            pltpu.VMEM((MB, D), jnp.float32),
            pltpu.SemaphoreType.DMA((6,)),
            pltpu.SemaphoreType.DMA,
        ],
        compiler_params=pltpu.CompilerParams(collective_id=0),
    )(P)


# device time: 357694 ns/iter; 2.5082x vs baseline; 2.5082x over previous
import jax
import jax.numpy as jnp
from jax import lax
from jax.experimental import pallas as pl
from jax.experimental.pallas import tpu as pltpu

M = 4096
D = 4096
K = 8192
MB = M // 4
NC = 16
CW = D // NC
HALF = CW // 2

MESH = pl.DeviceIdType.MESH


def kernel(dy, W):

    def body(dy_ref, w_ref, out_ref, dy_vmem, w_vmem, pc, yrecv,
             dy_sem, w_sems, store_sems, ysd, yrc, z1sd, z1rc,
             x1sd, x1rc, x2sd, x2rc, z2sd, z2rc, credit):
        xi = lax.axis_index("x")
        yi = lax.axis_index("y")
        zi = lax.axis_index("z")
        bb = 2 * xi + zi
        bz = 2 * xi + (1 - zi)
        bx = 2 * (1 - xi) + zi
        y_nbr = (xi, 1 - yi, zi)
        z_nbr = (xi, yi, 1 - zi)
        x_nbr = (1 - xi, yi, zi)

        rows_own = pl.ds(bb * MB, MB)
        rows_bz = pl.ds(bz * MB, MB)
        rows_bx = pl.ds(bx * MB, MB)

        def slot(k):
            return lax.rem(k, 2) if not isinstance(k, int) else k % 2

        def cols(k):
            return pl.ds(k * CW, CW)

        def colsA(k):
            return pl.ds(k * CW, HALF)

        def colsB(k):
            return pl.ds(k * CW + HALF, HALF)

        def rdma(src, dst, ssem, rsem, dev):
            return pltpu.make_async_remote_copy(
                src_ref=src, dst_ref=dst, send_sem=ssem, recv_sem=rsem,
                device_id=dev, device_id_type=MESH,
            )

        def w_copy(k):
            return pltpu.make_async_copy(
                w_ref.at[cols(k), :], w_vmem.at[slot(k)], w_sems.at[slot(k)]
            )

        def y_desc(k):
            return rdma(pc.at[slot(k)], yrecv.at[slot(k)],
                        ysd.at[k], yrc.at[k], y_nbr)

        def z1_desc(k):
            return rdma(pc.at[slot(k)], out_ref.at[rows_own, cols(k)],
                        z1sd.at[k], z1rc.at[k], z_nbr)

        def x1_desc(k):
            return rdma(pc.at[slot(k)], out_ref.at[rows_own, cols(k)],
                        x1sd.at[k], x1rc.at[k], x_nbr)

        def x2_desc(k):
            return rdma(out_ref.at[rows_bz, colsA(k)],
                        out_ref.at[rows_bz, colsA(k)],
                        x2sd.at[k], x2rc.at[k], x_nbr)

        def z2_desc(k):
            return rdma(out_ref.at[rows_bx, colsB(k)],
                        out_ref.at[rows_bx, colsB(k)],
                        z2sd.at[k], z2rc.at[k], z_nbr)

        def store_desc(k):
            return pltpu.make_async_copy(
                pc.at[slot(k)], out_ref.at[rows_own, cols(k)],
                store_sems.at[slot(k)]
            )

        def guard(k):
            z1_desc(k).wait_send()
            x1_desc(k).wait_send()
            store_desc(k).wait()

        def do_y_reduce(k):
            t = slot(k)
            y_desc(k).wait_send()
            y_desc(k).wait_recv()
            pc[t, :, :] = pc[t, :, :] + yrecv[t, :, :]
            pl.semaphore_signal(credit, inc=1, device_id=y_nbr,
                                device_id_type=MESH)
            store_desc(k).start()
            z1_desc(k).start()
            x1_desc(k).start()

        def do_zfwd(k):
            z1_desc(k).wait_recv()
            x2_desc(k).start()

        def do_xfwd(k):
            x1_desc(k).wait_recv()
            z2_desc(k).start()

        def do_drain(k):
            x2_desc(k).wait_recv()
            z2_desc(k).wait_recv()
            x2_desc(k).wait_send()
            z2_desc(k).wait_send()

        barrier = pltpu.get_barrier_semaphore()
        for nbr in [x_nbr, y_nbr, z_nbr]:
            pl.semaphore_signal(barrier, inc=1, device_id=nbr,
                                device_id_type=MESH)
        pl.semaphore_wait(barrier, 3)

        dy_copy = pltpu.make_async_copy(
            dy_ref.at[rows_own, :], dy_vmem, dy_sem
        )
        dy_copy.start()
        w_copy(0).start()
        dy_copy.wait()

        def step(c, carry):
            s = slot(c)

            @pl.when(c + 1 < NC)
            def _():
                w_copy(c + 1).start()

            @pl.when(c >= 2)
            def _():
                guard(c - 2)

            w_copy(c).wait()
            pc[s, :, :] = lax.dot_general(
                dy_vmem[:, :], w_vmem[s, :, :],
                (((1,), (1,)), ((), ())),
                preferred_element_type=jnp.float32,
            )

            @pl.when(c >= 2)
            def _():
                pl.semaphore_wait(credit, 1)

            y_desc(c).start()

            @pl.when(c >= 1)
            def _():
                do_y_reduce(c - 1)

            @pl.when(c >= 2)
            def _():
                do_zfwd(c - 2)
                do_xfwd(c - 2)

            @pl.when(c >= 3)
            def _():
                do_drain(c - 3)

            return carry

        lax.fori_loop(0, NC, step, 0)

        do_y_reduce(NC - 1)
        do_zfwd(NC - 2)
        do_xfwd(NC - 2)
        do_zfwd(NC - 1)
        do_xfwd(NC - 1)
        do_drain(NC - 3)
        do_drain(NC - 2)
        do_drain(NC - 1)
        guard(NC - 2)
        guard(NC - 1)
        pl.semaphore_wait(credit, 2)

    return pl.pallas_call(
        body,
        out_shape=jax.ShapeDtypeStruct((M, D), jnp.float32),
        in_specs=[
            pl.BlockSpec(memory_space=pl.ANY),
            pl.BlockSpec(memory_space=pl.ANY),
        ],
        out_specs=pl.BlockSpec(memory_space=pl.ANY),
        scratch_shapes=[
            pltpu.VMEM((MB, K), jnp.float32),
            pltpu.VMEM((2, CW, K), jnp.float32),
            pltpu.VMEM((2, MB, CW), jnp.float32),
            pltpu.VMEM((2, MB, CW), jnp.float32),
            pltpu.SemaphoreType.DMA,
            pltpu.SemaphoreType.DMA((2,)),
            pltpu.SemaphoreType.DMA((2,)),
            pltpu.SemaphoreType.DMA((NC,)),
            pltpu.SemaphoreType.DMA((NC,)),
            pltpu.SemaphoreType.DMA((NC,)),
            pltpu.SemaphoreType.DMA((NC,)),
            pltpu.SemaphoreType.DMA((NC,)),
            pltpu.SemaphoreType.DMA((NC,)),
            pltpu.SemaphoreType.DMA((NC,)),
            pltpu.SemaphoreType.DMA((NC,)),
            pltpu.SemaphoreType.DMA((NC,)),
            pltpu.SemaphoreType.DMA((NC,)),
            pltpu.SemaphoreType.REGULAR,
        ],
        compiler_params=pltpu.CompilerParams(
            collective_id=0, vmem_limit_bytes=63 * 1024 * 1024
        ),
    )(dy, W)
